# baseline (speedup 1.0000x reference)
import functools

import jax
import jax.numpy as jnp
from jax import lax
from jax.experimental import pallas as pl
from jax.experimental.pallas import tpu as pltpu
from jax.experimental.pallas import tpu_sc as plsc

MAXLEN = 8000
HEAD_DIM = 64
SEQ = 2048
N = SEQ * SEQ

NUM_CORES = 2
NUM_SUBCORES = 16
NUM_WORKERS = NUM_CORES * NUM_SUBCORES

IDX_MINOR = 128
ROWS_PER_STEP = 2
CHUNK = ROWS_PER_STEP * IDX_MINOR
NBUF = 2
GROUP_ROWS = ROWS_PER_STEP * NBUF
IDX_ROWS = N // IDX_MINOR
ROWS_PER_WORKER = IDX_ROWS // NUM_WORKERS
STEPS = ROWS_PER_WORKER // ROWS_PER_STEP
GROUPS = STEPS // NBUF
PER_WORKER_OUT = N // NUM_WORKERS


def _sc_body(idx_hbm, table_hbm, out_hbm, idx_v, rows_v, isems, gsems, wsems):
    wid = lax.axis_index("s") * NUM_CORES + lax.axis_index("c")
    row_base = wid * ROWS_PER_WORKER
    out_base = wid * PER_WORKER_OUT

    def out_slice(i):
        flat = out_base + i * CHUNK
        return out_hbm.at[flat // SEQ, pl.ds(flat % SEQ, CHUNK)]

    def idx_fetch(g, p):
        pltpu.async_copy(
            idx_hbm.at[pl.ds(row_base + g * GROUP_ROWS, GROUP_ROWS)],
            idx_v.at[p], isems[p])

    def idx_wait(p):
        pltpu.make_async_copy(
            idx_hbm.at[pl.ds(0, GROUP_ROWS)], idx_v.at[p], isems[p]).wait()

    def drain_write(i, b):
        pltpu.make_async_copy(rows_v.at[b], out_slice(i), wsems[b]).wait()

    def group(g, p):
        idx_wait(p)

        @pl.when(g + 1 < GROUPS)
        def _():
            idx_fetch(g + 1, 1 - p)

        for j in range(GROUP_ROWS):
            for k in range(IDX_MINOR // 16):
                v = idx_v[p, j, pl.ds(k * 16, 16)]
                v = jnp.minimum(jnp.maximum(v, -MAXLEN), MAXLEN - 1) + MAXLEN
                idx_v[p, j, pl.ds(k * 16, 16)] = v

        gathers = []
        for b in range(NBUF):
            i = g * NBUF + b
            @pl.when(g > 0)
            def _():
                drain_write(i, b)
            gathers.append([
                pltpu.async_copy(
                    table_hbm.at[idx_v.at[p, b * ROWS_PER_STEP + j]],
                    rows_v.at[b, pl.ds(j * IDX_MINOR, IDX_MINOR)],
                    gsems[b])
                for j in range(ROWS_PER_STEP)
            ])
        for b in range(NBUF):
            i = g * NBUF + b
            for cp in gathers[b]:
                cp.wait()
            pltpu.async_copy(rows_v.at[b], out_slice(i), wsems[b])

    def pair(t, c):
        group(2 * t, 0)
        group(2 * t + 1, 1)
        return c

    idx_fetch(0, 0)
    lax.fori_loop(0, GROUPS // 2, pair, 0)
    for b in range(NBUF):
        drain_write((GROUPS - 1) * NBUF + b, b)


@functools.partial(
    pl.kernel,
    out_type=jax.ShapeDtypeStruct((SEQ, SEQ, 2 * HEAD_DIM), jnp.float32),
    mesh=plsc.VectorSubcoreMesh(core_axis_name="c", subcore_axis_name="s"),
    scratch_types=[
        pltpu.VMEM((2, GROUP_ROWS, IDX_MINOR), jnp.int32),
        pltpu.VMEM((NBUF, CHUNK, 2 * HEAD_DIM), jnp.float32),
        [pltpu.SemaphoreType.DMA] * 2,
        [pltpu.SemaphoreType.DMA] * NBUF,
        [pltpu.SemaphoreType.DMA] * NBUF,
    ],
    compiler_params=pltpu.CompilerParams(use_tc_tiling_on_sc=True),
)
def _sc_gather(idx_hbm, table_hbm, out_hbm, idx_v, rows_v, isems, gsems, wsems):
    _sc_body(idx_hbm, table_hbm, out_hbm, idx_v, rows_v, isems, gsems, wsems)


def kernel(pos_seq, pe_k):
    idx2d = pos_seq.astype(jnp.int32).reshape(IDX_ROWS, IDX_MINOR)
    table = jnp.pad(pe_k, ((0, 0), (0, HEAD_DIM)))
    return _sc_gather(idx2d, table)[:, :, :HEAD_DIM]
